# baseline (speedup 1.0000x reference)
import functools

import jax
import jax.numpy as jnp
from jax import lax
from jax.experimental import pallas as pl
from jax.experimental.pallas import tpu as pltpu
from jax.experimental.pallas import tpu_sc as plsc

_N = 10000
_F = 128
_E = 320000
_CHUNK = 128
_TILES = 16
_CPT = 160
_NCHUNK = _CPT * _TILES
_EPAD = _NCHUNK * _CHUNK
_ACC_ROWS = _N + _CHUNK
_ROWS_MAIN = 624
_ZROWS = 48
_GRP = 8
_NGRP = _CPT // _GRP


def _seg_body(x_hbm, idx_hbm, out_hbm, ev0, ev1,
              rows0, rows1, zbuf, acc, gsem0, gsem1, isem0, isem1):
    c = lax.axis_index("c")
    s = lax.axis_index("s")

    zv = jnp.zeros((16,), jnp.float32)

    def zstore(i, carry):
        zbuf[i // 8, pl.ds((i % 8) * 16, 16)] = zv
        return carry

    lax.fori_loop(0, _ZROWS * 8, zstore, 0)

    row0 = s * _ROWS_MAIN + 8 * jnp.minimum(s, 2)

    def zcopy(k, carry):
        pltpu.sync_copy(zbuf, acc.at[pl.ds(row0 + k * _ZROWS, _ZROWS)])
        return carry

    lax.fori_loop(0, _ROWS_MAIN // _ZROWS, zcopy, 0)

    @pl.when(s < 2)
    def _():
        pltpu.sync_copy(zbuf.at[pl.ds(0, 8)], acc.at[pl.ds(row0 + _ROWS_MAIN, 8)])

    base = c * _NCHUNK + s
    plsc.subcore_barrier()

    pltpu.async_copy(idx_hbm.at[pl.ds(base * 2, 2)], ev0, isem0)
    pltpu.async_copy(idx_hbm.at[pl.ds((base + _TILES) * 2, 2)], ev1, isem1)

    def do_chunk(g, ev, rws, gsem, isem, prefetch):
        pltpu.make_async_copy(idx_hbm.at[pl.ds(base * 2, 2)], ev, isem).wait()
        pltpu.async_copy(x_hbm.at[ev.at[1]], rws, gsem).wait()
        pltpu.sync_copy(rws, acc.at[ev.at[0]], add=True)
        if prefetch:
            pltpu.async_copy(
                idx_hbm.at[pl.ds((base + (g + 2) * _TILES) * 2, 2)], ev, isem)

    def pair_body(gg, carry):
        do_chunk(gg * 2, ev0, rows0, gsem0, isem0, True)
        do_chunk(gg * 2 + 1, ev1, rows1, gsem1, isem1, True)
        return carry

    lax.fori_loop(0, _CPT // 2 - 1, pair_body, 0)
    do_chunk(_CPT - 2, ev0, rows0, gsem0, isem0, False)
    do_chunk(_CPT - 1, ev1, rows1, gsem1, isem1, False)
    plsc.subcore_barrier()

    pltpu.sync_copy(acc.at[pl.ds(row0, _ROWS_MAIN)],
                    out_hbm.at[c, pl.ds(row0, _ROWS_MAIN)])

    @pl.when(s < 2)
    def _():
        pltpu.sync_copy(acc.at[pl.ds(row0 + _ROWS_MAIN, 8)],
                        out_hbm.at[c, pl.ds(row0 + _ROWS_MAIN, 8)])


def _make_seg():
    mesh = plsc.VectorSubcoreMesh(core_axis_name="c", subcore_axis_name="s")
    return pl.kernel(
        _seg_body,
        out_type=jax.ShapeDtypeStruct((2, _N, _F), jnp.float32),
        mesh=mesh,
        scratch_types=[
            pltpu.VMEM((2, _CHUNK), jnp.int32),
            pltpu.VMEM((2, _CHUNK), jnp.int32),
            pltpu.VMEM((_CHUNK, _F), jnp.float32),
            pltpu.VMEM((_CHUNK, _F), jnp.float32),
            pltpu.VMEM((_ZROWS, _F), jnp.float32),
            pltpu.VMEM_SHARED((_ACC_ROWS, _F), jnp.float32),
            pltpu.SemaphoreType.DMA,
            pltpu.SemaphoreType.DMA,
            pltpu.SemaphoreType.DMA,
            pltpu.SemaphoreType.DMA,
        ],
        name="segment_sums_sc",
    )


def _dense_body(x_ref, nsum_ref, psum_ref, wn_ref, wp_ref, b_ref, g_ref,
                beta_ref, gk_ref, grk_ref, gb_ref, o_ref):
    x = x_ref[...]
    aggre = jnp.dot(nsum_ref[...], wn_ref[...], preferred_element_type=jnp.float32)
    aggre = aggre + jnp.dot(psum_ref[...], wp_ref[...], preferred_element_type=jnp.float32)
    aggre = aggre + b_ref[...] + x
    a = jnp.maximum(aggre, 0.0)
    mean = jnp.mean(a, axis=0, keepdims=True)
    var = jnp.mean((a - mean) * (a - mean), axis=0, keepdims=True)
    a = (a - mean) / jnp.sqrt(var + 1e-3) * g_ref[...] + beta_ref[...]
    mx = jnp.dot(a, gk_ref[...], preferred_element_type=jnp.float32) + gb_ref[0:1, :]
    mi = jnp.dot(x, grk_ref[...], preferred_element_type=jnp.float32) + gb_ref[1:2, :]
    z = jax.nn.sigmoid(mx[:, :_F] + mi[:, :_F])
    r = jax.nn.sigmoid(mx[:, _F:2 * _F] + mi[:, _F:2 * _F])
    h = jnp.tanh(mx[:, 2 * _F:] + r * mi[:, 2 * _F:])
    o_ref[...] = z * x + (1.0 - z) * h


def _make_dense(interpret=False):
    return pl.pallas_call(
        _dense_body,
        out_shape=jax.ShapeDtypeStruct((_N, _F), jnp.float32),
        interpret=interpret,
        name="dense_tail_tc",
    )


@functools.cache
def _get_seg():
    return _make_seg()


@functools.cache
def _get_dense():
    return _make_dense()


def kernel(x, pairs_prev, pairs_next, w_next, w_prev, b, bn_gamma, bn_beta,
           gru_kernel, gru_rec_kernel, gru_bias):
    npad = _EPAD - _E
    dpad = _N + (jnp.arange(npad, dtype=jnp.int32) % _CHUNK)
    spad = jnp.zeros((npad,), jnp.int32)
    dst = jnp.concatenate(
        [pairs_prev[:, 0], dpad, pairs_next[:, 0], dpad]).reshape(2 * _NCHUNK, _CHUNK)
    src = jnp.concatenate(
        [pairs_prev[:, 1], spad, pairs_next[:, 1], spad]).reshape(2 * _NCHUNK, _CHUNK)
    idx = jnp.stack([dst, src], axis=1).reshape(4 * _NCHUNK, _CHUNK)
    sums = _get_seg()(x, idx)
    prev_sumx = sums[0]
    next_sumx = sums[1]
    return _get_dense()(x, next_sumx, prev_sumx, w_next, w_prev, b,
                        bn_gamma.reshape(1, _F), bn_beta.reshape(1, _F),
                        gru_kernel, gru_rec_kernel, gru_bias)


# speedup vs baseline: 2.4196x; 2.4196x over previous
import jax
import jax.numpy as jnp
from jax import lax
from jax.experimental import pallas as pl
from jax.experimental.pallas import tpu as pltpu
from jax.experimental.pallas import tpu_sc as plsc

_N = 10000
_F = 128
_E = 320000
_CHUNK = 128
_NCHUNK = _E // _CHUNK
_TILES = 16
_ROWS_MAIN = 624
_ZROWS = 104


def _seg_body(x_hbm, dst_hbm, src_hbm, out_hbm, dst_v0, src_v0, dst_v1, src_v1,
              rows_v, zbuf, acc, sem, isem0, isem1):
    c = lax.axis_index("c")
    s = lax.axis_index("s")

    row0 = s * _ROWS_MAIN + 8 * jnp.minimum(s, 2)

    zv = jnp.zeros((16,), jnp.float32)

    def zstore(i, carry):
        zbuf[i // 8, pl.ds((i % 8) * 16, 16)] = zv
        return carry

    lax.fori_loop(0, _ZROWS * 8, zstore, 0)

    def zcopy(k, carry):
        pltpu.sync_copy(zbuf, acc.at[pl.ds(row0 + k * _ZROWS, _ZROWS)])
        return carry

    lax.fori_loop(0, _ROWS_MAIN // _ZROWS, zcopy, 0)

    @pl.when(s < 2)
    def _():
        pltpu.sync_copy(zbuf.at[pl.ds(0, 8)], acc.at[pl.ds(row0 + _ROWS_MAIN, 8)])

    plsc.subcore_barrier()

    def issue_idx(k, dv, sv, isem):
        base = c * _E + k * _CHUNK
        pltpu.async_copy(dst_hbm.at[pl.ds(base, _CHUNK)], dv, isem)
        pltpu.async_copy(src_hbm.at[pl.ds(base, _CHUNK)], sv, isem)

    issue_idx(s, dst_v0, src_v0, isem0)
    issue_idx(_TILES + s, dst_v1, src_v1, isem1)

    def do_chunk(g, dv, sv, isem):
        k = g * _TILES + s

        @pl.when(k < _NCHUNK)
        def _():
            base = c * _E + k * _CHUNK
            pltpu.make_async_copy(dst_hbm.at[pl.ds(base, _CHUNK)], dv, isem).wait()
            pltpu.make_async_copy(src_hbm.at[pl.ds(base, _CHUNK)], sv, isem).wait()
            pltpu.async_copy(x_hbm.at[sv], rows_v, sem).wait()
            pltpu.sync_copy(rows_v, acc.at[dv], add=True)

            @pl.when(k + 2 * _TILES < _NCHUNK)
            def _():
                issue_idx(k + 2 * _TILES, dv, sv, isem)

        return carry_unused

    carry_unused = 0

    def pair_body(gg, carry):
        do_chunk(gg * 2, dst_v0, src_v0, isem0)
        do_chunk(gg * 2 + 1, dst_v1, src_v1, isem1)
        return carry

    npair = ((_NCHUNK + _TILES - 1) // _TILES + 1) // 2
    lax.fori_loop(0, npair, pair_body, 0)
    plsc.subcore_barrier()

    pltpu.sync_copy(acc.at[pl.ds(row0, _ROWS_MAIN)],
                    out_hbm.at[c, pl.ds(row0, _ROWS_MAIN)])

    @pl.when(s < 2)
    def _():
        pltpu.sync_copy(acc.at[pl.ds(row0 + _ROWS_MAIN, 8)],
                        out_hbm.at[c, pl.ds(row0 + _ROWS_MAIN, 8)])


def _make_seg():
    mesh = plsc.VectorSubcoreMesh(core_axis_name="c", subcore_axis_name="s")
    return pl.kernel(
        _seg_body,
        out_type=jax.ShapeDtypeStruct((2, _N, _F), jnp.float32),
        mesh=mesh,
        scratch_types=[
            pltpu.VMEM((_CHUNK,), jnp.int32),
            pltpu.VMEM((_CHUNK,), jnp.int32),
            pltpu.VMEM((_CHUNK,), jnp.int32),
            pltpu.VMEM((_CHUNK,), jnp.int32),
            pltpu.VMEM((_CHUNK, _F), jnp.float32),
            pltpu.VMEM((_ZROWS, _F), jnp.float32),
            pltpu.VMEM_SHARED((_N, _F), jnp.float32),
            pltpu.SemaphoreType.DMA,
            pltpu.SemaphoreType.DMA,
            pltpu.SemaphoreType.DMA,
        ],
        name="segment_sums_sc",
    )


def _dense_body(x_ref, nsum_ref, psum_ref, wn_ref, wp_ref, b_ref, g_ref,
                beta_ref, gk_ref, grk_ref, gb_ref, o_ref):
    x = x_ref[...]
    aggre = jnp.dot(nsum_ref[...], wn_ref[...], preferred_element_type=jnp.float32)
    aggre = aggre + jnp.dot(psum_ref[...], wp_ref[...], preferred_element_type=jnp.float32)
    aggre = aggre + b_ref[...] + x
    a = jnp.maximum(aggre, 0.0)
    mean = jnp.mean(a, axis=0, keepdims=True)
    var = jnp.mean((a - mean) * (a - mean), axis=0, keepdims=True)
    a = (a - mean) / jnp.sqrt(var + 1e-3) * g_ref[...] + beta_ref[...]
    mx = jnp.dot(a, gk_ref[...], preferred_element_type=jnp.float32) + gb_ref[0:1, :]
    mi = jnp.dot(x, grk_ref[...], preferred_element_type=jnp.float32) + gb_ref[1:2, :]
    z = jax.nn.sigmoid(mx[:, :_F] + mi[:, :_F])
    r = jax.nn.sigmoid(mx[:, _F:2 * _F] + mi[:, _F:2 * _F])
    h = jnp.tanh(mx[:, 2 * _F:] + r * mi[:, 2 * _F:])
    o_ref[...] = z * x + (1.0 - z) * h


def _make_dense(interpret=False):
    return pl.pallas_call(
        _dense_body,
        out_shape=jax.ShapeDtypeStruct((_N, _F), jnp.float32),
        interpret=interpret,
        name="dense_tail_tc",
    )


import functools


@functools.cache
def _get_seg():
    return _make_seg()


@functools.cache
def _get_dense():
    return _make_dense()


def kernel(x, pairs_prev, pairs_next, w_next, w_prev, b, bn_gamma, bn_beta,
           gru_kernel, gru_rec_kernel, gru_bias):
    dst = jnp.concatenate([pairs_prev[:, 0], pairs_next[:, 0]])
    src = jnp.concatenate([pairs_prev[:, 1], pairs_next[:, 1]])
    sums = _get_seg()(x, dst, src)
    prev_sumx = sums[0]
    next_sumx = sums[1]
    return _get_dense()(x, next_sumx, prev_sumx, w_next, w_prev, b,
                  bn_gamma.reshape(1, _F), bn_beta.reshape(1, _F),
                  gru_kernel, gru_rec_kernel, gru_bias)


# speedup vs baseline: 3.2838x; 1.3571x over previous
import jax
import jax.numpy as jnp
from jax import lax
from jax.experimental import pallas as pl
from jax.experimental.pallas import tpu as pltpu
from jax.experimental.pallas import tpu_sc as plsc

_N = 10000
_F = 128
_E = 320000
_CHUNK = 128
_NCHUNK = _E // _CHUNK
_TILES = 16
_ROWS_MAIN = 624
_ZROWS = 104


def _seg_body(x_hbm, dst_hbm, src_hbm, out_hbm, dst_v0, src_v0, dst_v1, src_v1,
              rows0, rows1, zbuf, acc, gsem0, gsem1, isem0, isem1):
    c = lax.axis_index("c")
    s = lax.axis_index("s")

    row0 = s * _ROWS_MAIN + 8 * jnp.minimum(s, 2)

    zv = jnp.zeros((16,), jnp.float32)

    def zstore(i, carry):
        zbuf[i // 8, pl.ds((i % 8) * 16, 16)] = zv
        return carry

    lax.fori_loop(0, _ZROWS * 8, zstore, 0)

    def zcopy(k, carry):
        pltpu.sync_copy(zbuf, acc.at[pl.ds(row0 + k * _ZROWS, _ZROWS)])
        return carry

    lax.fori_loop(0, _ROWS_MAIN // _ZROWS, zcopy, 0)

    @pl.when(s < 2)
    def _():
        pltpu.sync_copy(zbuf.at[pl.ds(0, 8)], acc.at[pl.ds(row0 + _ROWS_MAIN, 8)])

    plsc.subcore_barrier()

    def issue_idx(k, dv, sv, isem):
        base = c * _E + k * _CHUNK
        pltpu.async_copy(dst_hbm.at[pl.ds(base, _CHUNK)], dv, isem)
        pltpu.async_copy(src_hbm.at[pl.ds(base, _CHUNK)], sv, isem)

    def wait_idx(dv, sv, isem):
        pltpu.make_async_copy(dst_hbm.at[pl.ds(0, _CHUNK)], dv, isem).wait()
        pltpu.make_async_copy(src_hbm.at[pl.ds(0, _CHUNK)], sv, isem).wait()

    issue_idx(s, dst_v0, src_v0, isem0)
    issue_idx(_TILES + s, dst_v1, src_v1, isem1)
    wait_idx(dst_v0, src_v0, isem0)
    pltpu.async_copy(x_hbm.at[src_v0], rows0, gsem0)

    bufs = ((dst_v0, src_v0, rows0, gsem0, isem0),
            (dst_v1, src_v1, rows1, gsem1, isem1))

    def do_chunk(g, p):
        dv, sv, rws, gsem, isem = bufs[p]
        dvq, svq, rwsq, gsemq, isemq = bufs[1 - p]
        k = g * _TILES + s

        @pl.when(k + _TILES < _NCHUNK)
        def _():
            wait_idx(dvq, svq, isemq)
            pltpu.async_copy(x_hbm.at[svq], rwsq, gsemq)

        @pl.when(k < _NCHUNK)
        def _():
            pltpu.make_async_copy(x_hbm.at[sv], rws, gsem).wait()
            pltpu.sync_copy(rws, acc.at[dv], add=True)

            @pl.when(k + 2 * _TILES < _NCHUNK)
            def _():
                issue_idx(k + 2 * _TILES, dv, sv, isem)

    def pair_body(gg, carry):
        do_chunk(gg * 2, 0)
        do_chunk(gg * 2 + 1, 1)
        return carry

    npair = ((_NCHUNK + _TILES - 1) // _TILES + 1) // 2
    lax.fori_loop(0, npair, pair_body, 0)
    plsc.subcore_barrier()

    pltpu.sync_copy(acc.at[pl.ds(row0, _ROWS_MAIN)],
                    out_hbm.at[c, pl.ds(row0, _ROWS_MAIN)])

    @pl.when(s < 2)
    def _():
        pltpu.sync_copy(acc.at[pl.ds(row0 + _ROWS_MAIN, 8)],
                        out_hbm.at[c, pl.ds(row0 + _ROWS_MAIN, 8)])


def _make_seg():
    mesh = plsc.VectorSubcoreMesh(core_axis_name="c", subcore_axis_name="s")
    return pl.kernel(
        _seg_body,
        out_type=jax.ShapeDtypeStruct((2, _N, _F), jnp.float32),
        mesh=mesh,
        scratch_types=[
            pltpu.VMEM((_CHUNK,), jnp.int32),
            pltpu.VMEM((_CHUNK,), jnp.int32),
            pltpu.VMEM((_CHUNK,), jnp.int32),
            pltpu.VMEM((_CHUNK,), jnp.int32),
            pltpu.VMEM((_CHUNK, _F), jnp.float32),
            pltpu.VMEM((_CHUNK, _F), jnp.float32),
            pltpu.VMEM((_ZROWS, _F), jnp.float32),
            pltpu.VMEM_SHARED((_N, _F), jnp.float32),
            pltpu.SemaphoreType.DMA,
            pltpu.SemaphoreType.DMA,
            pltpu.SemaphoreType.DMA,
            pltpu.SemaphoreType.DMA,
        ],
        name="segment_sums_sc",
    )


def _dense_body(x_ref, nsum_ref, psum_ref, wn_ref, wp_ref, b_ref, g_ref,
                beta_ref, gk_ref, grk_ref, gb_ref, o_ref):
    x = x_ref[...]
    aggre = jnp.dot(nsum_ref[...], wn_ref[...], preferred_element_type=jnp.float32)
    aggre = aggre + jnp.dot(psum_ref[...], wp_ref[...], preferred_element_type=jnp.float32)
    aggre = aggre + b_ref[...] + x
    a = jnp.maximum(aggre, 0.0)
    mean = jnp.mean(a, axis=0, keepdims=True)
    var = jnp.mean((a - mean) * (a - mean), axis=0, keepdims=True)
    a = (a - mean) / jnp.sqrt(var + 1e-3) * g_ref[...] + beta_ref[...]
    mx = jnp.dot(a, gk_ref[...], preferred_element_type=jnp.float32) + gb_ref[0:1, :]
    mi = jnp.dot(x, grk_ref[...], preferred_element_type=jnp.float32) + gb_ref[1:2, :]
    z = jax.nn.sigmoid(mx[:, :_F] + mi[:, :_F])
    r = jax.nn.sigmoid(mx[:, _F:2 * _F] + mi[:, _F:2 * _F])
    h = jnp.tanh(mx[:, 2 * _F:] + r * mi[:, 2 * _F:])
    o_ref[...] = z * x + (1.0 - z) * h


def _make_dense(interpret=False):
    return pl.pallas_call(
        _dense_body,
        out_shape=jax.ShapeDtypeStruct((_N, _F), jnp.float32),
        interpret=interpret,
        name="dense_tail_tc",
    )


import functools


@functools.cache
def _get_seg():
    return _make_seg()


@functools.cache
def _get_dense():
    return _make_dense()


def kernel(x, pairs_prev, pairs_next, w_next, w_prev, b, bn_gamma, bn_beta,
           gru_kernel, gru_rec_kernel, gru_bias):
    dst = jnp.concatenate([pairs_prev[:, 0], pairs_next[:, 0]])
    src = jnp.concatenate([pairs_prev[:, 1], pairs_next[:, 1]])
    sums = _get_seg()(x, dst, src)
    prev_sumx = sums[0]
    next_sumx = sums[1]
    return _get_dense()(x, next_sumx, prev_sumx, w_next, w_prev, b,
                  bn_gamma.reshape(1, _F), bn_beta.reshape(1, _F),
                  gru_kernel, gru_rec_kernel, gru_bias)


# speedup vs baseline: 3.6762x; 1.1195x over previous
import jax
import jax.numpy as jnp
from jax import lax
from jax.experimental import pallas as pl
from jax.experimental.pallas import tpu as pltpu
from jax.experimental.pallas import tpu_sc as plsc

_N = 10000
_F = 128
_E = 320000
_CHUNK = 128
_NCHUNK = _E // _CHUNK
_TILES = 16
_ROWS_MAIN = 624
_ZROWS = 104


def _seg_body(x_hbm, dst_hbm, src_hbm, out_hbm, dst_v0, src_v0, dst_v1, src_v1,
              rows0, rows1, zbuf, acc, gsem0, gsem1, isem0, isem1, ssem0, ssem1):
    c = lax.axis_index("c")
    s = lax.axis_index("s")

    row0 = s * _ROWS_MAIN + 8 * jnp.minimum(s, 2)

    zv = jnp.zeros((16,), jnp.float32)

    def zstore(i, carry):
        zbuf[i // 8, pl.ds((i % 8) * 16, 16)] = zv
        return carry

    lax.fori_loop(0, _ZROWS * 8, zstore, 0)

    def zcopy(k, carry):
        pltpu.sync_copy(zbuf, acc.at[pl.ds(row0 + k * _ZROWS, _ZROWS)])
        return carry

    lax.fori_loop(0, _ROWS_MAIN // _ZROWS, zcopy, 0)

    @pl.when(s < 2)
    def _():
        pltpu.sync_copy(zbuf.at[pl.ds(0, 8)], acc.at[pl.ds(row0 + _ROWS_MAIN, 8)])

    plsc.subcore_barrier()

    def issue_idx(k, dv, sv, isem):
        base = c * _E + k * _CHUNK
        pltpu.async_copy(dst_hbm.at[pl.ds(base, _CHUNK)], dv, isem)
        pltpu.async_copy(src_hbm.at[pl.ds(base, _CHUNK)], sv, isem)

    def wait_idx(dv, sv, isem):
        pltpu.make_async_copy(dst_hbm.at[pl.ds(0, _CHUNK)], dv, isem).wait()
        pltpu.make_async_copy(src_hbm.at[pl.ds(0, _CHUNK)], sv, isem).wait()

    issue_idx(s, dst_v0, src_v0, isem0)
    issue_idx(_TILES + s, dst_v1, src_v1, isem1)
    wait_idx(dst_v0, src_v0, isem0)
    pltpu.async_copy(x_hbm.at[src_v0], rows0, gsem0)

    bufs = ((dst_v0, src_v0, rows0, gsem0, isem0, ssem0),
            (dst_v1, src_v1, rows1, gsem1, isem1, ssem1))

    def do_chunk(g, p):
        dv, sv, rws, gsem, isem, ssem = bufs[p]
        dvq, svq, rwsq, gsemq, isemq, ssemq = bufs[1 - p]
        k = g * _TILES + s

        @pl.when((k >= _TILES) & (k - _TILES < _NCHUNK))
        def _():
            pltpu.make_async_copy(rwsq, acc.at[dvq], ssemq).wait()

        @pl.when(k + _TILES < _NCHUNK)
        def _():
            wait_idx(dvq, svq, isemq)
            pltpu.async_copy(x_hbm.at[svq], rwsq, gsemq)

        @pl.when(k < _NCHUNK)
        def _():
            pltpu.make_async_copy(x_hbm.at[sv], rws, gsem).wait()
            pltpu.async_copy(rws, acc.at[dv], ssem, add=True)

            @pl.when(k + 2 * _TILES < _NCHUNK)
            def _():
                issue_idx(k + 2 * _TILES, dv, sv, isem)

    def pair_body(gg, carry):
        do_chunk(gg * 2, 0)
        do_chunk(gg * 2 + 1, 1)
        return carry

    npair = ((_NCHUNK + _TILES - 1) // _TILES + 1) // 2
    lax.fori_loop(0, npair, pair_body, 0)
    last_k = (2 * npair - 1) * _TILES + s

    @pl.when(last_k < _NCHUNK)
    def _():
        pltpu.make_async_copy(rows1, acc.at[dst_v1], ssem1).wait()

    plsc.subcore_barrier()

    pltpu.sync_copy(acc.at[pl.ds(row0, _ROWS_MAIN)],
                    out_hbm.at[c, pl.ds(row0, _ROWS_MAIN)])

    @pl.when(s < 2)
    def _():
        pltpu.sync_copy(acc.at[pl.ds(row0 + _ROWS_MAIN, 8)],
                        out_hbm.at[c, pl.ds(row0 + _ROWS_MAIN, 8)])


def _make_seg():
    mesh = plsc.VectorSubcoreMesh(core_axis_name="c", subcore_axis_name="s")
    return pl.kernel(
        _seg_body,
        out_type=jax.ShapeDtypeStruct((2, _N, _F), jnp.float32),
        mesh=mesh,
        scratch_types=[
            pltpu.VMEM((_CHUNK,), jnp.int32),
            pltpu.VMEM((_CHUNK,), jnp.int32),
            pltpu.VMEM((_CHUNK,), jnp.int32),
            pltpu.VMEM((_CHUNK,), jnp.int32),
            pltpu.VMEM((_CHUNK, _F), jnp.float32),
            pltpu.VMEM((_CHUNK, _F), jnp.float32),
            pltpu.VMEM((_ZROWS, _F), jnp.float32),
            pltpu.VMEM_SHARED((_N, _F), jnp.float32),
            pltpu.SemaphoreType.DMA,
            pltpu.SemaphoreType.DMA,
            pltpu.SemaphoreType.DMA,
            pltpu.SemaphoreType.DMA,
            pltpu.SemaphoreType.DMA,
            pltpu.SemaphoreType.DMA,
        ],
        name="segment_sums_sc",
    )


def _dense_body(x_ref, nsum_ref, psum_ref, wn_ref, wp_ref, b_ref, g_ref,
                beta_ref, gk_ref, grk_ref, gb_ref, o_ref):
    x = x_ref[...]
    aggre = jnp.dot(nsum_ref[...], wn_ref[...], preferred_element_type=jnp.float32)
    aggre = aggre + jnp.dot(psum_ref[...], wp_ref[...], preferred_element_type=jnp.float32)
    aggre = aggre + b_ref[...] + x
    a = jnp.maximum(aggre, 0.0)
    mean = jnp.mean(a, axis=0, keepdims=True)
    var = jnp.mean((a - mean) * (a - mean), axis=0, keepdims=True)
    a = (a - mean) / jnp.sqrt(var + 1e-3) * g_ref[...] + beta_ref[...]
    mx = jnp.dot(a, gk_ref[...], preferred_element_type=jnp.float32) + gb_ref[0:1, :]
    mi = jnp.dot(x, grk_ref[...], preferred_element_type=jnp.float32) + gb_ref[1:2, :]
    z = jax.nn.sigmoid(mx[:, :_F] + mi[:, :_F])
    r = jax.nn.sigmoid(mx[:, _F:2 * _F] + mi[:, _F:2 * _F])
    h = jnp.tanh(mx[:, 2 * _F:] + r * mi[:, 2 * _F:])
    o_ref[...] = z * x + (1.0 - z) * h


def _make_dense(interpret=False):
    return pl.pallas_call(
        _dense_body,
        out_shape=jax.ShapeDtypeStruct((_N, _F), jnp.float32),
        interpret=interpret,
        name="dense_tail_tc",
    )


import functools


@functools.cache
def _get_seg():
    return _make_seg()


@functools.cache
def _get_dense():
    return _make_dense()


def kernel(x, pairs_prev, pairs_next, w_next, w_prev, b, bn_gamma, bn_beta,
           gru_kernel, gru_rec_kernel, gru_bias):
    dst = jnp.concatenate([pairs_prev[:, 0], pairs_next[:, 0]])
    src = jnp.concatenate([pairs_prev[:, 1], pairs_next[:, 1]])
    sums = _get_seg()(x, dst, src)
    prev_sumx = sums[0]
    next_sumx = sums[1]
    return _get_dense()(x, next_sumx, prev_sumx, w_next, w_prev, b,
                  bn_gamma.reshape(1, _F), bn_beta.reshape(1, _F),
                  gru_kernel, gru_rec_kernel, gru_bias)


# speedup vs baseline: 3.6762x; 1.0000x over previous
import jax
import jax.numpy as jnp
from jax import lax
from jax.experimental import pallas as pl
from jax.experimental.pallas import tpu as pltpu
from jax.experimental.pallas import tpu_sc as plsc

_N = 10000
_F = 128
_E = 320000
_CHUNK = 128
_NCHUNK = _E // _CHUNK
_TILES = 16
_ROWS_MAIN = 624
_ZROWS = 104


def _seg_body(x_hbm, dst_hbm, src_hbm, out_hbm, dst_v0, src_v0, dst_v1, src_v1,
              ov0, ov1, rows0, rows1, zbuf, acc, gsem0, gsem1, isem0, isem1,
              ssem0, ssem1):
    c = lax.axis_index("c")
    s = lax.axis_index("s")

    row0 = s * _ROWS_MAIN + 8 * jnp.minimum(s, 2)

    zv = jnp.zeros((16,), jnp.float32)

    def zstore(i, carry):
        zbuf[i // 8, pl.ds((i % 8) * 16, 16)] = zv
        return carry

    lax.fori_loop(0, _ZROWS * 8, zstore, 0)

    def zcopy(k, carry):
        pltpu.sync_copy(zbuf, acc.at[pl.ds(row0 + k * _ZROWS, _ZROWS)])
        return carry

    lax.fori_loop(0, _ROWS_MAIN // _ZROWS, zcopy, 0)

    @pl.when(s < 2)
    def _():
        pltpu.sync_copy(zbuf.at[pl.ds(0, 8)], acc.at[pl.ds(row0 + _ROWS_MAIN, 8)])

    plsc.subcore_barrier()

    def issue_idx(k, dv, sv, isem):
        base = c * _E + k * _CHUNK
        pltpu.async_copy(dst_hbm.at[pl.ds(base, _CHUNK)], dv, isem)
        pltpu.async_copy(src_hbm.at[pl.ds(base, _CHUNK)], sv, isem)

    def wait_idx(dv, sv, isem):
        pltpu.make_async_copy(dst_hbm.at[pl.ds(0, _CHUNK)], dv, isem).wait()
        pltpu.make_async_copy(src_hbm.at[pl.ds(0, _CHUNK)], sv, isem).wait()

    issue_idx(s, dst_v0, src_v0, isem0)
    issue_idx(_TILES + s, dst_v1, src_v1, isem1)
    wait_idx(dst_v0, src_v0, isem0)
    pltpu.async_copy(x_hbm.at[src_v0], rows0, gsem0)

    bufs = ((dst_v0, src_v0, ov0, rows0, gsem0, isem0, ssem0),
            (dst_v1, src_v1, ov1, rows1, gsem1, isem1, ssem1))

    def do_chunk(g, p):
        dv, sv, ov, rws, gsem, isem, ssem = bufs[p]
        dvq, svq, ovq, rwsq, gsemq, isemq, ssemq = bufs[1 - p]
        k = g * _TILES + s

        @pl.when((k >= _TILES) & (k - _TILES < _NCHUNK))
        def _():
            pltpu.make_async_copy(rwsq, acc.at[ovq], ssemq).wait()

        @pl.when(k + _TILES < _NCHUNK)
        def _():
            wait_idx(dvq, svq, isemq)
            pltpu.async_copy(x_hbm.at[svq], rwsq, gsemq)

        @pl.when(k < _NCHUNK)
        def _():
            pltpu.make_async_copy(x_hbm.at[sv], rws, gsem).wait()
            for t in range(_CHUNK // 16):
                ov[pl.ds(t * 16, 16)] = dv[pl.ds(t * 16, 16)]
            pltpu.async_copy(rws, acc.at[ov], ssem, add=True)

            @pl.when(k + 2 * _TILES < _NCHUNK)
            def _():
                issue_idx(k + 2 * _TILES, dv, sv, isem)

    def pair_body(gg, carry):
        do_chunk(gg * 2, 0)
        do_chunk(gg * 2 + 1, 1)
        return carry

    npair = ((_NCHUNK + _TILES - 1) // _TILES + 1) // 2
    lax.fori_loop(0, npair, pair_body, 0)
    last_k = (2 * npair - 1) * _TILES + s

    @pl.when(last_k < _NCHUNK)
    def _():
        pltpu.make_async_copy(rows1, acc.at[ov1], ssem1).wait()

    plsc.subcore_barrier()

    pltpu.sync_copy(acc.at[pl.ds(row0, _ROWS_MAIN)],
                    out_hbm.at[c, pl.ds(row0, _ROWS_MAIN)])

    @pl.when(s < 2)
    def _():
        pltpu.sync_copy(acc.at[pl.ds(row0 + _ROWS_MAIN, 8)],
                        out_hbm.at[c, pl.ds(row0 + _ROWS_MAIN, 8)])


def _make_seg():
    mesh = plsc.VectorSubcoreMesh(core_axis_name="c", subcore_axis_name="s")
    return pl.kernel(
        _seg_body,
        out_type=jax.ShapeDtypeStruct((2, _N, _F), jnp.float32),
        mesh=mesh,
        scratch_types=[
            pltpu.VMEM((_CHUNK,), jnp.int32),
            pltpu.VMEM((_CHUNK,), jnp.int32),
            pltpu.VMEM((_CHUNK,), jnp.int32),
            pltpu.VMEM((_CHUNK,), jnp.int32),
            pltpu.VMEM((_CHUNK,), jnp.int32),
            pltpu.VMEM((_CHUNK,), jnp.int32),
            pltpu.VMEM((_CHUNK, _F), jnp.float32),
            pltpu.VMEM((_CHUNK, _F), jnp.float32),
            pltpu.VMEM((_ZROWS, _F), jnp.float32),
            pltpu.VMEM_SHARED((_N, _F), jnp.float32),
            pltpu.SemaphoreType.DMA,
            pltpu.SemaphoreType.DMA,
            pltpu.SemaphoreType.DMA,
            pltpu.SemaphoreType.DMA,
            pltpu.SemaphoreType.DMA,
            pltpu.SemaphoreType.DMA,
        ],
        name="segment_sums_sc",
    )


def _dense_body(x_ref, nsum_ref, psum_ref, wn_ref, wp_ref, b_ref, g_ref,
                beta_ref, gk_ref, grk_ref, gb_ref, o_ref):
    x = x_ref[...]
    aggre = jnp.dot(nsum_ref[...], wn_ref[...], preferred_element_type=jnp.float32)
    aggre = aggre + jnp.dot(psum_ref[...], wp_ref[...], preferred_element_type=jnp.float32)
    aggre = aggre + b_ref[...] + x
    a = jnp.maximum(aggre, 0.0)
    mean = jnp.mean(a, axis=0, keepdims=True)
    var = jnp.mean((a - mean) * (a - mean), axis=0, keepdims=True)
    a = (a - mean) / jnp.sqrt(var + 1e-3) * g_ref[...] + beta_ref[...]
    mx = jnp.dot(a, gk_ref[...], preferred_element_type=jnp.float32) + gb_ref[0:1, :]
    mi = jnp.dot(x, grk_ref[...], preferred_element_type=jnp.float32) + gb_ref[1:2, :]
    z = jax.nn.sigmoid(mx[:, :_F] + mi[:, :_F])
    r = jax.nn.sigmoid(mx[:, _F:2 * _F] + mi[:, _F:2 * _F])
    h = jnp.tanh(mx[:, 2 * _F:] + r * mi[:, 2 * _F:])
    o_ref[...] = z * x + (1.0 - z) * h


def _make_dense(interpret=False):
    return pl.pallas_call(
        _dense_body,
        out_shape=jax.ShapeDtypeStruct((_N, _F), jnp.float32),
        interpret=interpret,
        name="dense_tail_tc",
    )


import functools


@functools.cache
def _get_seg():
    return _make_seg()


@functools.cache
def _get_dense():
    return _make_dense()


def kernel(x, pairs_prev, pairs_next, w_next, w_prev, b, bn_gamma, bn_beta,
           gru_kernel, gru_rec_kernel, gru_bias):
    dst = jnp.concatenate([pairs_prev[:, 0], pairs_next[:, 0]])
    src = jnp.concatenate([pairs_prev[:, 1], pairs_next[:, 1]])
    sums = _get_seg()(x, dst, src)
    prev_sumx = sums[0]
    next_sumx = sums[1]
    return _get_dense()(x, next_sumx, prev_sumx, w_next, w_prev, b,
                  bn_gamma.reshape(1, _F), bn_beta.reshape(1, _F),
                  gru_kernel, gru_rec_kernel, gru_bias)
